# baseline (speedup 1.0000x reference)
import functools

import jax
import jax.numpy as jnp
from jax import lax
from jax.experimental import pallas as pl
from jax.experimental.pallas import tpu as pltpu
from jax.experimental.pallas import tpu_sc as plsc

N = 10000
D = 128
E = 320000
NW = 32
EPW = 10240
EPAD = NW * EPW
C = 128
NCHUNK = EPW // C
NACC = 10240
RPT = NACC // 16
HALF = NCHUNK // 2

_mesh = plsc.VectorSubcoreMesh(core_axis_name="c", subcore_axis_name="s")


def _edge_body(hs_hbm, src_hbm, dst_hbm, out_hbm, acc_sh, srcv, dstv,
               rows0, rows1, gs0, gs1, ss0, ss1, isem):
    cid = lax.axis_index("c")
    sid = lax.axis_index("s")
    wid = sid * 2 + cid
    rows = (rows0, rows1)
    gs = (gs0, gs1)
    ss = (ss0, ss1)

    def _z(i, _):
        rows0[i // 8, pl.ds((i % 8) * 16, 16)] = jnp.zeros((16,), jnp.float32)
        return 0
    lax.fori_loop(0, C * 8, _z, 0)
    def _zc(j, _):
        pltpu.sync_copy(rows0, acc_sh.at[pl.ds((sid * 5 + j) * C, C)])
        return 0
    lax.fori_loop(0, 5, _zc, 0)
    plsc.subcore_barrier()

    def gather(j, b):
        return pltpu.async_copy(hs_hbm.at[srcv.at[j]], rows[b], gs[b])

    def scatter(j, b):
        return pltpu.async_copy(rows[b], acc_sh.at[dstv.at[j]], ss[b],
                                add=True)

    for h in range(NCHUNK // HALF):
        hb = wid * NCHUNK + h * HALF
        pltpu.async_copy(src_hbm.at[pl.ds(hb, HALF)], srcv, isem).wait()
        pltpu.async_copy(dst_hbm.at[pl.ds(hb, HALF)], dstv, isem).wait()
        gather(0, 0)
        gather(1, 1)

        def body(k, _):
            i = k * 2
            for b in range(2):
                j = i + b
                pltpu.make_async_copy(hs_hbm.at[srcv.at[j]], rows[b],
                                      gs[b]).wait()
                scatter(j, b)
                pltpu.make_async_copy(rows[b], acc_sh.at[dstv.at[j]],
                                      ss[b]).wait()
                gather(j + 2, b)
            return 0

        lax.fori_loop(0, (HALF - 2) // 2, body, 0)
        for b in range(2):
            j = HALF - 2 + b
            pltpu.make_async_copy(hs_hbm.at[srcv.at[j]], rows[b], gs[b]).wait()
            scatter(j, b)
            pltpu.make_async_copy(rows[b], acc_sh.at[dstv.at[j]], ss[b]).wait()

    plsc.subcore_barrier()
    pltpu.sync_copy(acc_sh.at[pl.ds(sid * RPT, RPT)],
                    out_hbm.at[cid, pl.ds(sid * RPT, RPT)])


@functools.partial(jax.jit, donate_argnums=())
def _edge_pass(hs, src2d, dst2d):
    return pl.kernel(
        _edge_body,
        out_type=jax.ShapeDtypeStruct((2, NACC, D), jnp.float32),
        mesh=_mesh,
        scratch_types=[
            pltpu.VMEM_SHARED((NACC, D), jnp.float32),
            pltpu.VMEM((HALF, C), jnp.int32),
            pltpu.VMEM((HALF, C), jnp.int32),
            pltpu.VMEM((C, D), jnp.float32),
            pltpu.VMEM((C, D), jnp.float32),
            pltpu.SemaphoreType.DMA,
            pltpu.SemaphoreType.DMA,
            pltpu.SemaphoreType.DMA,
            pltpu.SemaphoreType.DMA,
            pltpu.SemaphoreType.DMA,
        ],
    )(hs, src2d, dst2d)


DEB = 64


def _tcdeg_body(hi_ref, lo_ref, acc_ref):
    @pl.when(pl.program_id(0) == 0)
    def _():
        acc_ref[...] = jnp.zeros_like(acc_ref)
    kcol = lax.broadcasted_iota(jnp.int16, (128, 1), 0)
    tot = jnp.zeros((128, 128), jnp.float32)
    for j in range(DEB):
        a = (hi_ref[j:j + 1, :] == kcol).astype(jnp.bfloat16)
        bt = (lo_ref[j:j + 1, :] == kcol).astype(jnp.bfloat16)
        tot = tot + lax.dot_general(a, bt, (((1,), (1,)), ((), ())),
                                    preferred_element_type=jnp.float32)
    acc_ref[...] += tot


def _deg_pass(dhi, dlo):
    return pl.pallas_call(
        _tcdeg_body,
        grid=(EPAD // (DEB * 128),),
        in_specs=[pl.BlockSpec((DEB, 128), lambda i: (i, 0)),
                  pl.BlockSpec((DEB, 128), lambda i: (i, 0))],
        out_specs=pl.BlockSpec((128, 128), lambda i: (0, 0)),
        out_shape=jax.ShapeDtypeStruct((128, 128), jnp.float32),
    )(dhi, dlo)


BR = 400
GRID = N // BR


def _dinv_block(degs):
    return lax.rsqrt(degs[...] + 1.0)


def _tc1_body(x_ref, w_ref, degs_ref, hs_ref):
    dinv = _dinv_block(degs_ref)
    h = jnp.dot(x_ref[...], w_ref[...], precision=lax.Precision.HIGHEST,
                preferred_element_type=jnp.float32)
    hs_ref[...] = h * dinv


def _tc1(x, W1, degs):
    return pl.pallas_call(
        _tc1_body,
        grid=(GRID,),
        in_specs=[
            pl.BlockSpec((BR, D), lambda i: (i, 0)),
            pl.BlockSpec((D, D), lambda i: (0, 0)),
            pl.BlockSpec((BR, 1), lambda i: (i, 0)),
        ],
        out_specs=pl.BlockSpec((BR, D), lambda i: (i, 0)),
        out_shape=jax.ShapeDtypeStruct((N, D), jnp.float32),
    )(x, W1, degs)


def _tc2_body(acc_ref, hs_ref, degs_ref, b_ref, w_ref, hs2_ref):
    dinv = _dinv_block(degs_ref)
    z = dinv * (acc_ref[0] + acc_ref[1] + hs_ref[...]) + b_ref[...]
    z = jnp.maximum(z, 0.0)
    h2 = jnp.dot(z, w_ref[...], precision=lax.Precision.HIGHEST,
                 preferred_element_type=jnp.float32)
    hs2_ref[...] = h2 * dinv


def _tc2(acc1, hs1, degs, b1, W2):
    return pl.pallas_call(
        _tc2_body,
        grid=(GRID,),
        in_specs=[
            pl.BlockSpec((2, BR, D), lambda i: (0, i, 0)),
            pl.BlockSpec((BR, D), lambda i: (i, 0)),
            pl.BlockSpec((BR, 1), lambda i: (i, 0)),
            pl.BlockSpec((1, D), lambda i: (0, 0)),
            pl.BlockSpec((D, D), lambda i: (0, 0)),
        ],
        out_specs=pl.BlockSpec((BR, D), lambda i: (i, 0)),
        out_shape=jax.ShapeDtypeStruct((N, D), jnp.float32),
    )(acc1, hs1, degs, b1, W2)


def _tc3_body(acc_ref, hs_ref, degs_ref, b_ref, out_ref):
    dinv = _dinv_block(degs_ref)
    out_ref[...] = dinv * (acc_ref[0] + acc_ref[1] + hs_ref[...]) + b_ref[...]


def _tc3(acc2, hs2, degs, b2):
    return pl.pallas_call(
        _tc3_body,
        grid=(GRID,),
        in_specs=[
            pl.BlockSpec((2, BR, D), lambda i: (0, i, 0)),
            pl.BlockSpec((BR, D), lambda i: (i, 0)),
            pl.BlockSpec((BR, 1), lambda i: (i, 0)),
            pl.BlockSpec((1, D), lambda i: (0, 0)),
        ],
        out_specs=pl.BlockSpec((BR, D), lambda i: (i, 0)),
        out_shape=jax.ShapeDtypeStruct((N, D), jnp.float32),
    )(acc2, hs2, degs, b2)


def kernel(x, edge_index, W1, b1, W2, b2):
    src = edge_index[0].astype(jnp.int32)
    dst = edge_index[1].astype(jnp.int32)
    pad = EPAD - E
    ar = jnp.arange(pad, dtype=jnp.int32)
    src_all = jnp.concatenate([src, (ar * 997) % N])
    dst_all = jnp.concatenate([dst, N + (ar % 128)])

    src2d = src_all.reshape(EPAD // C, C)
    dst2d = dst_all.reshape(EPAD // C, C)
    dhi = jnp.right_shift(dst2d, 7).astype(jnp.int16)
    dlo = jnp.bitwise_and(dst2d, 127).astype(jnp.int16)
    deg2d = _deg_pass(dhi, dlo)
    degs = deg2d.reshape(128 * 128, 1)
    hs1 = _tc1(x, W1, degs)
    acc1 = _edge_pass(hs1, src2d, dst2d)
    hs2 = _tc2(acc1, hs1, degs, b1.reshape(1, D), W2)
    acc2 = _edge_pass(hs2, src2d, dst2d)
    return _tc3(acc2, hs2, degs, b2.reshape(1, D))


# speedup vs baseline: 1.0335x; 1.0335x over previous
import functools

import jax
import jax.numpy as jnp
from jax import lax
from jax.experimental import pallas as pl
from jax.experimental.pallas import tpu as pltpu
from jax.experimental.pallas import tpu_sc as plsc

N = 10000
D = 128
E = 320000
NW = 32
EPW = 10240
EPAD = NW * EPW
C = 128
NCHUNK = EPW // C
NACC = 10240
RPT = NACC // 16
HALF = NCHUNK // 2

_mesh = plsc.VectorSubcoreMesh(core_axis_name="c", subcore_axis_name="s")


def _edge_body(hs_hbm, src_hbm, dst_hbm, out_hbm, acc_sh, srcv, dstv,
               rows0, rows1, gs0, gs1, ss0, ss1, isem):
    cid = lax.axis_index("c")
    sid = lax.axis_index("s")
    wid = sid * 2 + cid
    rows = (rows0, rows1)
    gs = (gs0, gs1)
    ss = (ss0, ss1)

    def _z(i, _):
        rows0[i // 8, pl.ds((i % 8) * 16, 16)] = jnp.zeros((16,), jnp.float32)
        return 0
    lax.fori_loop(0, C * 8, _z, 0)
    def _zc(j, _):
        pltpu.sync_copy(rows0, acc_sh.at[pl.ds((sid * 5 + j) * C, C)])
        return 0
    lax.fori_loop(0, 5, _zc, 0)
    plsc.subcore_barrier()

    def gather(j, b):
        return pltpu.async_copy(hs_hbm.at[srcv.at[j]], rows[b], gs[b])

    def scatter(j, b):
        return pltpu.async_copy(rows[b], acc_sh.at[dstv.at[j]], ss[b],
                                add=True)

    for h in range(NCHUNK // HALF):
        hb = wid * NCHUNK + h * HALF
        pltpu.async_copy(src_hbm.at[pl.ds(hb, HALF)], srcv, isem).wait()
        pltpu.async_copy(dst_hbm.at[pl.ds(hb, HALF)], dstv, isem).wait()
        gather(0, 0)
        gather(1, 1)

        def body(k, _):
            i = k * 2
            for b in range(2):
                j = i + b
                pltpu.make_async_copy(hs_hbm.at[srcv.at[j]], rows[b],
                                      gs[b]).wait()
                scatter(j, b)
                pltpu.make_async_copy(rows[b], acc_sh.at[dstv.at[j]],
                                      ss[b]).wait()
                gather(j + 2, b)
            return 0

        lax.fori_loop(0, (HALF - 2) // 2, body, 0)
        for b in range(2):
            j = HALF - 2 + b
            pltpu.make_async_copy(hs_hbm.at[srcv.at[j]], rows[b], gs[b]).wait()
            scatter(j, b)
            pltpu.make_async_copy(rows[b], acc_sh.at[dstv.at[j]], ss[b]).wait()

    plsc.subcore_barrier()
    pltpu.sync_copy(acc_sh.at[pl.ds(sid * RPT, RPT)],
                    out_hbm.at[cid, pl.ds(sid * RPT, RPT)])


@functools.partial(jax.jit, donate_argnums=())
def _edge_pass(hs, src2d, dst2d):
    return pl.kernel(
        _edge_body,
        out_type=jax.ShapeDtypeStruct((2, NACC, D), jnp.float32),
        mesh=_mesh,
        scratch_types=[
            pltpu.VMEM_SHARED((NACC, D), jnp.float32),
            pltpu.VMEM((HALF, C), jnp.int32),
            pltpu.VMEM((HALF, C), jnp.int32),
            pltpu.VMEM((C, D), jnp.float32),
            pltpu.VMEM((C, D), jnp.float32),
            pltpu.SemaphoreType.DMA,
            pltpu.SemaphoreType.DMA,
            pltpu.SemaphoreType.DMA,
            pltpu.SemaphoreType.DMA,
            pltpu.SemaphoreType.DMA,
        ],
    )(hs, src2d, dst2d)


DEB = 64


def _tcdeg_body(hi_ref, lo_ref, acc_ref):
    @pl.when(pl.program_id(0) == 0)
    def _():
        acc_ref[...] = jnp.zeros_like(acc_ref)
    kcol = lax.broadcasted_iota(jnp.int32, (128, 1), 0)
    tot = jnp.zeros((128, 128), jnp.float32)
    for j in range(DEB):
        a = (hi_ref[j:j + 1, :] == kcol).astype(jnp.bfloat16)
        bt = (lo_ref[j:j + 1, :] == kcol).astype(jnp.bfloat16)
        tot = tot + lax.dot_general(a, bt, (((1,), (1,)), ((), ())),
                                    preferred_element_type=jnp.float32)
    acc_ref[...] += tot


def _deg_pass(dhi, dlo):
    return pl.pallas_call(
        _tcdeg_body,
        grid=(EPAD // (DEB * 128),),
        in_specs=[pl.BlockSpec((DEB, 128), lambda i: (i, 0)),
                  pl.BlockSpec((DEB, 128), lambda i: (i, 0))],
        out_specs=pl.BlockSpec((128, 128), lambda i: (0, 0)),
        out_shape=jax.ShapeDtypeStruct((128, 128), jnp.float32),
    )(dhi, dlo)


BR = 400
GRID = N // BR


def _dinv_block(degs):
    return lax.rsqrt(degs[...] + 1.0)


def _tc1_body(x_ref, w_ref, degs_ref, hs_ref):
    dinv = _dinv_block(degs_ref)
    h = jnp.dot(x_ref[...], w_ref[...], precision=lax.Precision.HIGHEST,
                preferred_element_type=jnp.float32)
    hs_ref[...] = h * dinv


def _tc1(x, W1, degs):
    return pl.pallas_call(
        _tc1_body,
        grid=(GRID,),
        in_specs=[
            pl.BlockSpec((BR, D), lambda i: (i, 0)),
            pl.BlockSpec((D, D), lambda i: (0, 0)),
            pl.BlockSpec((BR, 1), lambda i: (i, 0)),
        ],
        out_specs=pl.BlockSpec((BR, D), lambda i: (i, 0)),
        out_shape=jax.ShapeDtypeStruct((N, D), jnp.float32),
    )(x, W1, degs)


def _tc2_body(acc_ref, hs_ref, degs_ref, b_ref, w_ref, hs2_ref):
    dinv = _dinv_block(degs_ref)
    z = dinv * (acc_ref[0] + acc_ref[1] + hs_ref[...]) + b_ref[...]
    z = jnp.maximum(z, 0.0)
    h2 = jnp.dot(z, w_ref[...], precision=lax.Precision.HIGHEST,
                 preferred_element_type=jnp.float32)
    hs2_ref[...] = h2 * dinv


def _tc2(acc1, hs1, degs, b1, W2):
    return pl.pallas_call(
        _tc2_body,
        grid=(GRID,),
        in_specs=[
            pl.BlockSpec((2, BR, D), lambda i: (0, i, 0)),
            pl.BlockSpec((BR, D), lambda i: (i, 0)),
            pl.BlockSpec((BR, 1), lambda i: (i, 0)),
            pl.BlockSpec((1, D), lambda i: (0, 0)),
            pl.BlockSpec((D, D), lambda i: (0, 0)),
        ],
        out_specs=pl.BlockSpec((BR, D), lambda i: (i, 0)),
        out_shape=jax.ShapeDtypeStruct((N, D), jnp.float32),
    )(acc1, hs1, degs, b1, W2)


def _tc3_body(acc_ref, hs_ref, degs_ref, b_ref, out_ref):
    dinv = _dinv_block(degs_ref)
    out_ref[...] = dinv * (acc_ref[0] + acc_ref[1] + hs_ref[...]) + b_ref[...]


def _tc3(acc2, hs2, degs, b2):
    return pl.pallas_call(
        _tc3_body,
        grid=(GRID,),
        in_specs=[
            pl.BlockSpec((2, BR, D), lambda i: (0, i, 0)),
            pl.BlockSpec((BR, D), lambda i: (i, 0)),
            pl.BlockSpec((BR, 1), lambda i: (i, 0)),
            pl.BlockSpec((1, D), lambda i: (0, 0)),
        ],
        out_specs=pl.BlockSpec((BR, D), lambda i: (i, 0)),
        out_shape=jax.ShapeDtypeStruct((N, D), jnp.float32),
    )(acc2, hs2, degs, b2)


def kernel(x, edge_index, W1, b1, W2, b2):
    src = edge_index[0].astype(jnp.int32)
    dst = edge_index[1].astype(jnp.int32)
    pad = EPAD - E
    ar = jnp.arange(pad, dtype=jnp.int32)
    src_all = jnp.concatenate([src, (ar * 997) % N])
    dst_all = jnp.concatenate([dst, N + (ar % 128)])

    src2d = src_all.reshape(EPAD // C, C)
    dst2d = dst_all.reshape(EPAD // C, C)
    dhi = jnp.right_shift(dst2d, 7)
    dlo = jnp.bitwise_and(dst2d, 127)
    deg2d = _deg_pass(dhi, dlo)
    degs = deg2d.reshape(128 * 128, 1)
    hs1 = _tc1(x, W1, degs)
    acc1 = _edge_pass(hs1, src2d, dst2d)
    hs2 = _tc2(acc1, hs1, degs, b1.reshape(1, D), W2)
    acc2 = _edge_pass(hs2, src2d, dst2d)
    return _tc3(acc2, hs2, degs, b2.reshape(1, D))


# speedup vs baseline: 1.1209x; 1.0846x over previous
import functools

import jax
import jax.numpy as jnp
from jax import lax
from jax.experimental import pallas as pl
from jax.experimental.pallas import tpu as pltpu
from jax.experimental.pallas import tpu_sc as plsc

N = 10000
D = 128
E = 320000
NW = 32
EPW = 10240
EPAD = NW * EPW
C = 128
NCHUNK = EPW // C
NACC = 10240
RPT = NACC // 16
HALF = NCHUNK // 2

_mesh = plsc.VectorSubcoreMesh(core_axis_name="c", subcore_axis_name="s")


def _edge_body(hs_hbm, src_hbm, dst_hbm, out_hbm, acc_sh, srcv, dstv,
               rows0, rows1, gs0, gs1, ss0, ss1, isem):
    cid = lax.axis_index("c")
    sid = lax.axis_index("s")
    wid = sid * 2 + cid
    rows = (rows0, rows1)
    gs = (gs0, gs1)
    ss = (ss0, ss1)

    def _z(i, _):
        rows0[i // 8, pl.ds((i % 8) * 16, 16)] = jnp.zeros((16,), jnp.float32)
        return 0
    lax.fori_loop(0, C * 8, _z, 0)
    def _zc(j, _):
        pltpu.sync_copy(rows0, acc_sh.at[pl.ds((sid * 5 + j) * C, C)])
        return 0
    lax.fori_loop(0, 5, _zc, 0)
    plsc.subcore_barrier()

    def gather(j, b):
        return pltpu.async_copy(hs_hbm.at[srcv.at[j]], rows[b], gs[b])

    def scatter(j, b):
        return pltpu.async_copy(rows[b], acc_sh.at[dstv.at[j]], ss[b],
                                add=True)

    for h in range(NCHUNK // HALF):
        hb = wid * NCHUNK + h * HALF
        pltpu.async_copy(src_hbm.at[pl.ds(hb, HALF)], srcv, isem).wait()
        pltpu.async_copy(dst_hbm.at[pl.ds(hb, HALF)], dstv, isem).wait()
        gather(0, 0)
        gather(1, 1)

        def body(k, _):
            i = k * 2
            for b in range(2):
                j = i + b
                pltpu.make_async_copy(hs_hbm.at[srcv.at[j]], rows[b],
                                      gs[b]).wait()
                scatter(j, b)
                pltpu.make_async_copy(rows[b], acc_sh.at[dstv.at[j]],
                                      ss[b]).wait()
                gather(j + 2, b)
            return 0

        lax.fori_loop(0, (HALF - 2) // 2, body, 0)
        for b in range(2):
            j = HALF - 2 + b
            pltpu.make_async_copy(hs_hbm.at[srcv.at[j]], rows[b], gs[b]).wait()
            scatter(j, b)
            pltpu.make_async_copy(rows[b], acc_sh.at[dstv.at[j]], ss[b]).wait()

    plsc.subcore_barrier()
    pltpu.sync_copy(acc_sh.at[pl.ds(sid * RPT, RPT)],
                    out_hbm.at[cid, pl.ds(sid * RPT, RPT)])


@functools.partial(jax.jit, donate_argnums=())
def _edge_pass(hs, src2d, dst2d):
    return pl.kernel(
        _edge_body,
        out_type=jax.ShapeDtypeStruct((2, NACC, D), jnp.float32),
        mesh=_mesh,
        scratch_types=[
            pltpu.VMEM_SHARED((NACC, D), jnp.float32),
            pltpu.VMEM((HALF, C), jnp.int32),
            pltpu.VMEM((HALF, C), jnp.int32),
            pltpu.VMEM((C, D), jnp.float32),
            pltpu.VMEM((C, D), jnp.float32),
            pltpu.SemaphoreType.DMA,
            pltpu.SemaphoreType.DMA,
            pltpu.SemaphoreType.DMA,
            pltpu.SemaphoreType.DMA,
            pltpu.SemaphoreType.DMA,
        ],
    )(hs, src2d, dst2d)


DEB = 64


def _tcdeg_body(hi_ref, lo_ref, acc_ref):
    @pl.when(pl.program_id(0) == 0)
    def _():
        acc_ref[...] = jnp.zeros_like(acc_ref)
    kcol = lax.broadcasted_iota(jnp.int32, (128, 1), 0)
    tot = jnp.zeros((128, 128), jnp.float32)
    G = 8
    for g in range(DEB // G):
        a = jnp.concatenate(
            [(hi_ref[j:j + 1, :] == kcol).astype(jnp.bfloat16)
             for j in range(g * G, (g + 1) * G)], axis=1)
        bt = jnp.concatenate(
            [(lo_ref[j:j + 1, :] == kcol).astype(jnp.bfloat16)
             for j in range(g * G, (g + 1) * G)], axis=1)
        tot = tot + lax.dot_general(a, bt, (((1,), (1,)), ((), ())),
                                    preferred_element_type=jnp.float32)
    acc_ref[...] += tot


def _deg_pass(dhi, dlo):
    return pl.pallas_call(
        _tcdeg_body,
        grid=(EPAD // (DEB * 128),),
        in_specs=[pl.BlockSpec((DEB, 128), lambda i: (i, 0)),
                  pl.BlockSpec((DEB, 128), lambda i: (i, 0))],
        out_specs=pl.BlockSpec((128, 128), lambda i: (0, 0)),
        out_shape=jax.ShapeDtypeStruct((128, 128), jnp.float32),
    )(dhi, dlo)


BR = 400
GRID = N // BR


def _dinv_block(degs):
    return lax.rsqrt(degs[...] + 1.0)


def _tc1_body(x_ref, w_ref, degs_ref, hs_ref):
    dinv = _dinv_block(degs_ref)
    h = jnp.dot(x_ref[...], w_ref[...], precision=lax.Precision.HIGHEST,
                preferred_element_type=jnp.float32)
    hs_ref[...] = h * dinv


def _tc1(x, W1, degs):
    return pl.pallas_call(
        _tc1_body,
        grid=(GRID,),
        in_specs=[
            pl.BlockSpec((BR, D), lambda i: (i, 0)),
            pl.BlockSpec((D, D), lambda i: (0, 0)),
            pl.BlockSpec((BR, 1), lambda i: (i, 0)),
        ],
        out_specs=pl.BlockSpec((BR, D), lambda i: (i, 0)),
        out_shape=jax.ShapeDtypeStruct((N, D), jnp.float32),
    )(x, W1, degs)


def _tc2_body(acc_ref, hs_ref, degs_ref, b_ref, w_ref, hs2_ref):
    dinv = _dinv_block(degs_ref)
    z = dinv * (acc_ref[0] + acc_ref[1] + hs_ref[...]) + b_ref[...]
    z = jnp.maximum(z, 0.0)
    h2 = jnp.dot(z, w_ref[...], precision=lax.Precision.HIGHEST,
                 preferred_element_type=jnp.float32)
    hs2_ref[...] = h2 * dinv


def _tc2(acc1, hs1, degs, b1, W2):
    return pl.pallas_call(
        _tc2_body,
        grid=(GRID,),
        in_specs=[
            pl.BlockSpec((2, BR, D), lambda i: (0, i, 0)),
            pl.BlockSpec((BR, D), lambda i: (i, 0)),
            pl.BlockSpec((BR, 1), lambda i: (i, 0)),
            pl.BlockSpec((1, D), lambda i: (0, 0)),
            pl.BlockSpec((D, D), lambda i: (0, 0)),
        ],
        out_specs=pl.BlockSpec((BR, D), lambda i: (i, 0)),
        out_shape=jax.ShapeDtypeStruct((N, D), jnp.float32),
    )(acc1, hs1, degs, b1, W2)


def _tc3_body(acc_ref, hs_ref, degs_ref, b_ref, out_ref):
    dinv = _dinv_block(degs_ref)
    out_ref[...] = dinv * (acc_ref[0] + acc_ref[1] + hs_ref[...]) + b_ref[...]


def _tc3(acc2, hs2, degs, b2):
    return pl.pallas_call(
        _tc3_body,
        grid=(GRID,),
        in_specs=[
            pl.BlockSpec((2, BR, D), lambda i: (0, i, 0)),
            pl.BlockSpec((BR, D), lambda i: (i, 0)),
            pl.BlockSpec((BR, 1), lambda i: (i, 0)),
            pl.BlockSpec((1, D), lambda i: (0, 0)),
        ],
        out_specs=pl.BlockSpec((BR, D), lambda i: (i, 0)),
        out_shape=jax.ShapeDtypeStruct((N, D), jnp.float32),
    )(acc2, hs2, degs, b2)


def kernel(x, edge_index, W1, b1, W2, b2):
    src = edge_index[0].astype(jnp.int32)
    dst = edge_index[1].astype(jnp.int32)
    pad = EPAD - E
    ar = jnp.arange(pad, dtype=jnp.int32)
    src_all = jnp.concatenate([src, (ar * 997) % N])
    dst_all = jnp.concatenate([dst, N + (ar % 128)])

    src2d = src_all.reshape(EPAD // C, C)
    dst2d = dst_all.reshape(EPAD // C, C)
    dhi = jnp.right_shift(dst2d, 7)
    dlo = jnp.bitwise_and(dst2d, 127)
    deg2d = _deg_pass(dhi, dlo)
    degs = deg2d.reshape(128 * 128, 1)
    hs1 = _tc1(x, W1, degs)
    acc1 = _edge_pass(hs1, src2d, dst2d)
    hs2 = _tc2(acc1, hs1, degs, b1.reshape(1, D), W2)
    acc2 = _edge_pass(hs2, src2d, dst2d)
    return _tc3(acc2, hs2, degs, b2.reshape(1, D))
